# baseline (device time: 8667 ns/iter reference)
import jax
import jax.numpy as jnp
import numpy as np
from jax import lax
from jax.experimental import pallas as pl
from jax.experimental.pallas import tpu as pltpu

N_DEV = 4


def kernel(x):
    m, n = x.shape
    tri = jnp.asarray(np.tril(np.ones((m, m), np.float32)), dtype=jnp.bfloat16)

    def body(x_ref, tri_ref, out_ref, tot_ref, gather_ref, send_sems, recv_sems):
        my = lax.axis_index("i")

        barrier = pltpu.get_barrier_semaphore()
        for d in (1, 2, 3):
            pl.semaphore_signal(
                barrier, inc=1,
                device_id=((my + d) % N_DEV,),
                device_id_type=pl.DeviceIdType.MESH,
            )
        pl.semaphore_wait(barrier, 3)

        xv = x_ref[:, :]
        tot_ref[0, :] = jnp.sum(xv.astype(jnp.float32), axis=0)

        rdmas = []
        for d in (1, 2, 3):
            rdma = pltpu.make_async_remote_copy(
                src_ref=tot_ref,
                dst_ref=gather_ref.at[d - 1],
                send_sem=send_sems.at[d - 1],
                recv_sem=recv_sems.at[d - 1],
                device_id=((my + d) % N_DEV,),
                device_id_type=pl.DeviceIdType.MESH,
            )
            rdma.start()
            rdmas.append(rdma)

        cs = jax.lax.dot(
            tri_ref[:, :], xv.astype(jnp.bfloat16),
            preferred_element_type=jnp.float32,
        )

        offset = jnp.zeros((1, n), jnp.float32)
        for d in (1, 2, 3):
            rdmas[d - 1].wait_recv()
            src = (my - d) % N_DEV
            mask = jnp.where(src < my, 1.0, 0.0).astype(jnp.float32)
            offset = offset + gather_ref[d - 1, :, :] * mask

        out_ref[:, :] = (cs + offset).astype(jnp.bfloat16)

        for d in (1, 2, 3):
            rdmas[d - 1].wait_send()

    out_shape = jax.ShapeDtypeStruct((m, n), jnp.bfloat16)
    return pl.pallas_call(
        body,
        out_shape=out_shape,
        in_specs=[
            pl.BlockSpec(memory_space=pltpu.VMEM),
            pl.BlockSpec(memory_space=pltpu.VMEM),
        ],
        out_specs=pl.BlockSpec(memory_space=pltpu.VMEM),
        scratch_shapes=[
            pltpu.VMEM((1, n), jnp.float32),
            pltpu.VMEM((3, 1, n), jnp.float32),
            pltpu.SemaphoreType.DMA((3,)),
            pltpu.SemaphoreType.DMA((3,)),
        ],
        compiler_params=pltpu.CompilerParams(collective_id=0),
    )(x, tri)


# device time: 7235 ns/iter; 1.1979x vs baseline; 1.1979x over previous
import jax
import jax.numpy as jnp
import numpy as np
from jax import lax
from jax.experimental import pallas as pl
from jax.experimental.pallas import tpu as pltpu

N_DEV = 4


def kernel(x):
    m, n = x.shape
    def body(x_ref, out_ref, tot_ref, gather_ref, send_sems, recv_sems):
        my = lax.axis_index("i")

        barrier = pltpu.get_barrier_semaphore()
        for d in (1, 2, 3):
            pl.semaphore_signal(
                barrier, inc=1,
                device_id=((my + d) % N_DEV,),
                device_id_type=pl.DeviceIdType.MESH,
            )
        pl.semaphore_wait(barrier, 3)

        xv = x_ref[:, :]
        tot_ref[0, :] = jnp.sum(xv.astype(jnp.float32), axis=0)

        rdmas = []
        for d in (1, 2, 3):
            rdma = pltpu.make_async_remote_copy(
                src_ref=tot_ref,
                dst_ref=gather_ref.at[d - 1],
                send_sem=send_sems.at[d - 1],
                recv_sem=recv_sems.at[d - 1],
                device_id=((my + d) % N_DEV,),
                device_id_type=pl.DeviceIdType.MESH,
            )
            rdma.start()
            rdmas.append(rdma)

        row = lax.broadcasted_iota(jnp.int32, (m, m), 0)
        col = lax.broadcasted_iota(jnp.int32, (m, m), 1)
        tri = (row >= col).astype(jnp.bfloat16)
        cs = jax.lax.dot(
            tri, xv.astype(jnp.bfloat16),
            preferred_element_type=jnp.float32,
        )

        offset = jnp.zeros((1, n), jnp.float32)
        for d in (1, 2, 3):
            rdmas[d - 1].wait_recv()
            src = (my - d) % N_DEV
            mask = jnp.where(src < my, 1.0, 0.0).astype(jnp.float32)
            offset = offset + gather_ref[d - 1, :, :] * mask

        out_ref[:, :] = (cs + offset).astype(jnp.bfloat16)

        for d in (1, 2, 3):
            rdmas[d - 1].wait_send()

    out_shape = jax.ShapeDtypeStruct((m, n), jnp.bfloat16)
    return pl.pallas_call(
        body,
        out_shape=out_shape,
        in_specs=[pl.BlockSpec(memory_space=pltpu.VMEM)],
        out_specs=pl.BlockSpec(memory_space=pltpu.VMEM),
        scratch_shapes=[
            pltpu.VMEM((1, n), jnp.float32),
            pltpu.VMEM((3, 1, n), jnp.float32),
            pltpu.SemaphoreType.DMA((3,)),
            pltpu.SemaphoreType.DMA((3,)),
        ],
        compiler_params=pltpu.CompilerParams(collective_id=0),
    )(x)
